# baseline (speedup 1.0000x reference)
import jax
import jax.numpy as jnp
from jax.experimental import pallas as pl
from jax.experimental.pallas import tpu as pltpu

_K = 3
_J = 3
_ALPHA = 0.1
_NU = 0.25

_MB = 100000
_CDIM = 64
_QN = 392
_BLK = 2048
_NPAIR = 26
_NBLK = 2 * _NPAIR
_PADM = _BLK * _NBLK
_NCH = _BLK // 128
_TOPK = _K + _J


def _ce(a, b):
    return jnp.minimum(a, b), jnp.maximum(a, b)


def _sorted4(c0, c1, c2, c3):
    a0, a1 = _ce(c0, c1)
    b0, b1 = _ce(c2, c3)
    a0, b0 = _ce(a0, b0)
    a1, b1 = _ce(a1, b1)
    a1, b0 = _ce(a1, b0)
    return a0, a1, b0, b1


def _merge22(a0, a1, b0, b1):
    a0, b0 = _ce(a0, b0)
    a1, b1 = _ce(a1, b1)
    a1, b0 = _ce(a1, b0)
    return a0, a1, b0, b1


def _merge44_top6(a, b):
    e0, e1, e2, e3 = _merge22(a[0], a[2], b[0], b[2])
    o0, o1, o2, _ = _merge22(a[1], a[3], b[1], b[3])
    z1, z2 = _ce(o0, e1)
    z3, z4 = _ce(o1, e2)
    z5 = jnp.minimum(o2, e3)
    return e0, z1, z2, z3, z4, z5


def _merge66_top6(x, y):
    z = [jnp.minimum(x[i], y[5 - i]) for i in range(6)]
    z0, z3 = _ce(z[0], z[3])
    z1, z4 = _ce(z[1], z[4])
    z2, z5 = _ce(z[2], z[5])
    z0, z2 = _ce(z0, z2)
    z0, z1 = _ce(z0, z1)
    z1, z2 = _ce(z1, z2)
    z3, z5 = _ce(z3, z5)
    z3, z4 = _ce(z3, z4)
    z4, z5 = _ce(z4, z5)
    return z0, z1, z2, z3, z4, z5


def _matmul_block(xa, m_ref, lo, blk_idx, d_ref):
    m = m_ref[lo:lo + _BLK, :]
    mn = jnp.sum(m * m, axis=1, keepdims=True)
    row = blk_idx * _BLK + jax.lax.broadcasted_iota(jnp.int32, (_BLK, 1), 0)
    mn = jnp.where(row < _MB, mn, jnp.inf)
    ma = jnp.concatenate([m, mn], axis=1)
    d_ref[...] = jax.lax.dot_general(
        xa, ma, dimension_numbers=(((1,), (1,)), ((), ())),
        preferred_element_type=jnp.float32)


def _network(d_ref, cand_ref):
    def chunk(j):
        return d_ref[:, j * 128:(j + 1) * 128]

    def top6_of8(g):
        sa = _sorted4(chunk(8 * g), chunk(8 * g + 1),
                      chunk(8 * g + 2), chunk(8 * g + 3))
        sb = _sorted4(chunk(8 * g + 4), chunk(8 * g + 5),
                      chunk(8 * g + 6), chunk(8 * g + 7))
        return _merge44_top6(sa, sb)

    t = _merge66_top6(top6_of8(0), top6_of8(1))
    run = tuple(cand_ref[k] for k in range(_TOPK))
    new = _merge66_top6(t, run)
    for k in range(_TOPK):
        cand_ref[k] = new[k]


def _body(xa_ref, m_ref, r_ref, out_ref, cand_ref, da_ref, db_ref):
    i = pl.program_id(0)

    @pl.when(i == 0)
    def _init():
        cand_ref[...] = jnp.full((_TOPK, _QN, 128), jnp.inf, jnp.float32)
        db_ref[...] = jnp.full((_QN, _BLK), jnp.inf, jnp.float32)

    xa = xa_ref[...]
    _network(db_ref, cand_ref)
    _matmul_block(xa, m_ref, 0, 2 * i, da_ref)
    _network(da_ref, cand_ref)
    _matmul_block(xa, m_ref, _BLK, 2 * i + 1, db_ref)

    @pl.when(i == _NPAIR - 1)
    def _finish():
        cands = jnp.concatenate([cand_ref[k] for k in range(_TOPK)],
                                axis=1)
        iota = jax.lax.broadcasted_iota(jnp.int32, (_QN, _TOPK * 128), 1)
        xq = xa[:, 0:_CDIM]
        xn = jnp.sum(0.25 * (xq * xq), axis=1, keepdims=True)
        r2 = r_ref[0] * r_ref[0]
        att = jnp.float32(0.0)
        rep = jnp.float32(0.0)
        cur = cands
        for k in range(_TOPK):
            v = jnp.min(cur, axis=1, keepdims=True)
            sq = jnp.maximum(v + xn, 0.0)
            if k < _K:
                att = att + jnp.sum(jnp.maximum(sq - r2, 0.0))
            if k >= _J:
                rep = rep + jnp.sum(jnp.maximum(r2 - sq - _ALPHA, 0.0))
            if k < _TOPK - 1:
                midx = jnp.where(cur == v, iota, _TOPK * 128)
                jm = jnp.min(midx, axis=1, keepdims=True)
                cur = jnp.where(iota == jm, jnp.inf, cur)
        total = (att + rep) * ((1.0 / _NU) / (_QN * _K))
        out_ref[...] = jnp.full((1, 1), total, jnp.float32)


def kernel(phi_p, memory_bank, r):
    b, c, h, w = phi_p.shape
    x = jnp.transpose(phi_p, (0, 2, 3, 1)).reshape(b * h * w, c)
    xa = jnp.concatenate(
        [-2.0 * x, jnp.ones((b * h * w, 1), jnp.float32)], axis=1)
    mpad = jnp.pad(memory_bank, ((0, _PADM - _MB), (0, 0)))
    out = pl.pallas_call(
        _body,
        grid=(_NPAIR,),
        in_specs=[
            pl.BlockSpec((_QN, _CDIM + 1), lambda i: (0, 0)),
            pl.BlockSpec((2 * _BLK, _CDIM), lambda i: (i, 0)),
            pl.BlockSpec(memory_space=pltpu.SMEM),
        ],
        out_specs=pl.BlockSpec((1, 1), lambda i: (0, 0)),
        out_shape=jax.ShapeDtypeStruct((1, 1), jnp.float32),
        scratch_shapes=[
            pltpu.VMEM((_TOPK, _QN, 128), jnp.float32),
            pltpu.VMEM((_QN, _BLK), jnp.float32),
            pltpu.VMEM((_QN, _BLK), jnp.float32),
        ],
    )(xa, mpad, r)
    return out.reshape(())


# speedup vs baseline: 1.3084x; 1.3084x over previous
import jax
import jax.numpy as jnp
from jax.experimental import pallas as pl
from jax.experimental.pallas import tpu as pltpu

_K = 3
_J = 3
_ALPHA = 0.1
_NU = 0.25

_MB = 100000
_CDIM = 64
_QN = 392
_BLK = 2048
_NPAIR = 25
_NCH = _BLK // 128
_TOPK = _K + _J


def _ce(a, b):
    return jnp.minimum(a, b), jnp.maximum(a, b)


def _sorted4(c0, c1, c2, c3):
    a0, a1 = _ce(c0, c1)
    b0, b1 = _ce(c2, c3)
    a0, b0 = _ce(a0, b0)
    a1, b1 = _ce(a1, b1)
    a1, b0 = _ce(a1, b0)
    return a0, a1, b0, b1


def _merge22(a0, a1, b0, b1):
    a0, b0 = _ce(a0, b0)
    a1, b1 = _ce(a1, b1)
    a1, b0 = _ce(a1, b0)
    return a0, a1, b0, b1


def _merge44_top6(a, b):
    e0, e1, e2, e3 = _merge22(a[0], a[2], b[0], b[2])
    o0, o1, o2, _ = _merge22(a[1], a[3], b[1], b[3])
    z1, z2 = _ce(o0, e1)
    z3, z4 = _ce(o1, e2)
    z5 = jnp.minimum(o2, e3)
    return e0, z1, z2, z3, z4, z5


def _merge66_top6(x, y):
    z = [jnp.minimum(x[i], y[5 - i]) for i in range(6)]
    z0, z3 = _ce(z[0], z[3])
    z1, z4 = _ce(z[1], z[4])
    z2, z5 = _ce(z[2], z[5])
    z0, z2 = _ce(z0, z2)
    z0, z1 = _ce(z0, z1)
    z1, z2 = _ce(z1, z2)
    z3, z5 = _ce(z3, z5)
    z3, z4 = _ce(z3, z4)
    z4, z5 = _ce(z4, z5)
    return z0, z1, z2, z3, z4, z5


def _matmul_block(xa, m_ref, lo, blk_idx, d_ref):
    row = blk_idx * _BLK + jax.lax.broadcasted_iota(jnp.int32, (_BLK, 1), 0)
    m = jnp.where(row < _MB, m_ref[lo:lo + _BLK, :], 0.0)
    mn = jnp.sum(m * m, axis=1, keepdims=True)
    mn = jnp.where(row < _MB, mn, jnp.inf)
    ma = jnp.concatenate([m, mn], axis=1)
    d_ref[...] = jax.lax.dot_general(
        xa, ma, dimension_numbers=(((1,), (1,)), ((), ())),
        preferred_element_type=jnp.float32)


def _network(d_ref, cand_ref):
    def chunk(j):
        return d_ref[:, j * 128:(j + 1) * 128]

    def top6_of8(g):
        sa = _sorted4(chunk(8 * g), chunk(8 * g + 1),
                      chunk(8 * g + 2), chunk(8 * g + 3))
        sb = _sorted4(chunk(8 * g + 4), chunk(8 * g + 5),
                      chunk(8 * g + 6), chunk(8 * g + 7))
        return _merge44_top6(sa, sb)

    t = _merge66_top6(top6_of8(0), top6_of8(1))
    run = tuple(cand_ref[k] for k in range(_TOPK))
    new = _merge66_top6(t, run)
    for k in range(_TOPK):
        cand_ref[k] = new[k]


def _body(xa_ref, m_ref, r_ref, out_ref, cand_ref, da_ref, db_ref):
    i = pl.program_id(0)

    @pl.when(i == 0)
    def _init():
        cand_ref[...] = jnp.full((_TOPK, _QN, 128), jnp.inf, jnp.float32)
        db_ref[...] = jnp.full((_QN, _BLK), jnp.inf, jnp.float32)

    xa = xa_ref[...]
    _network(db_ref, cand_ref)
    _matmul_block(xa, m_ref, 0, 2 * i, da_ref)
    _network(da_ref, cand_ref)
    _matmul_block(xa, m_ref, _BLK, 2 * i + 1, db_ref)

    @pl.when(i == _NPAIR - 1)
    def _finish():
        cands = jnp.concatenate([cand_ref[k] for k in range(_TOPK)],
                                axis=1)
        iota = jax.lax.broadcasted_iota(jnp.int32, (_QN, _TOPK * 128), 1)
        xq = xa[:, 0:_CDIM]
        xn = jnp.sum(0.25 * (xq * xq), axis=1, keepdims=True)
        r2 = r_ref[0] * r_ref[0]
        att = jnp.float32(0.0)
        rep = jnp.float32(0.0)
        cur = cands
        for k in range(_TOPK):
            v = jnp.min(cur, axis=1, keepdims=True)
            sq = jnp.maximum(v + xn, 0.0)
            if k < _K:
                att = att + jnp.sum(jnp.maximum(sq - r2, 0.0))
            if k >= _J:
                rep = rep + jnp.sum(jnp.maximum(r2 - sq - _ALPHA, 0.0))
            if k < _TOPK - 1:
                midx = jnp.where(cur == v, iota, _TOPK * 128)
                jm = jnp.min(midx, axis=1, keepdims=True)
                cur = jnp.where(iota == jm, jnp.inf, cur)
        total = (att + rep) * ((1.0 / _NU) / (_QN * _K))
        out_ref[...] = jnp.full((1, 1), total, jnp.float32)


def kernel(phi_p, memory_bank, r):
    b, c, h, w = phi_p.shape
    x = jnp.transpose(phi_p, (0, 2, 3, 1)).reshape(b * h * w, c)
    xa = jnp.concatenate(
        [-2.0 * x, jnp.ones((b * h * w, 1), jnp.float32)], axis=1)
    out = pl.pallas_call(
        _body,
        grid=(_NPAIR,),
        in_specs=[
            pl.BlockSpec((_QN, _CDIM + 1), lambda i: (0, 0)),
            pl.BlockSpec((2 * _BLK, _CDIM), lambda i: (i, 0)),
            pl.BlockSpec(memory_space=pltpu.SMEM),
        ],
        out_specs=pl.BlockSpec((1, 1), lambda i: (0, 0)),
        out_shape=jax.ShapeDtypeStruct((1, 1), jnp.float32),
        scratch_shapes=[
            pltpu.VMEM((_TOPK, _QN, 128), jnp.float32),
            pltpu.VMEM((_QN, _BLK), jnp.float32),
            pltpu.VMEM((_QN, _BLK), jnp.float32),
        ],
    )(xa, memory_bank, r)
    return out.reshape(())
